# Initial kernel scaffold; baseline (speedup 1.0000x reference)
import functools

import jax
import jax.numpy as jnp
from jax import lax
from jax.experimental import pallas as pl
from jax.experimental.pallas import tpu as pltpu
from jax.experimental.pallas import tpu_sc as plsc

NC = 2
NS = 16
NW = NC * NS
CHUNK = 80
BLK = 2048

_mesh = functools.partial(
    plsc.VectorSubcoreMesh, core_axis_name="c", subcore_axis_name="s")


def _sc_degrees(ei, zeros1, n_pad, nchunk):
  slab_n = n_pad // NS

  @functools.partial(
      pl.kernel,
      out_type=jax.ShapeDtypeStruct((NC, 2, n_pad), jnp.float32),
      mesh=_mesh(),
      scratch_types=[
          pltpu.VMEM((2, nchunk, CHUNK), jnp.int32),
          pltpu.VMEM((CHUNK,), jnp.float32),
          pltpu.VMEM_SHARED((n_pad,), jnp.float32),
          pltpu.VMEM_SHARED((n_pad,), jnp.float32),
      ],
  )
  def k(ei_hbm, z_hbm, deg_hbm, idx_v, ones_v, dego_sh, degi_sh):
    cid = lax.axis_index("c")
    sid = lax.axis_index("s")
    w = cid * NS + sid
    pltpu.sync_copy(ei_hbm.at[0, w], idx_v.at[0])
    pltpu.sync_copy(ei_hbm.at[1, w], idx_v.at[1])
    for t in range(CHUNK // 16):
      ones_v[pl.ds(t * 16, 16)] = jnp.ones((16,), jnp.float32)
    slab = pl.ds(sid * slab_n, slab_n)
    pltpu.sync_copy(z_hbm.at[slab], dego_sh.at[slab])
    pltpu.sync_copy(z_hbm.at[slab], degi_sh.at[slab])
    plsc.subcore_barrier()

    def body(j, carry):
      pltpu.sync_copy(ones_v, dego_sh.at[idx_v.at[0, j]], add=True)
      pltpu.sync_copy(ones_v, degi_sh.at[idx_v.at[1, j]], add=True)
      return carry

    lax.fori_loop(0, nchunk, body, 0)
    plsc.subcore_barrier()
    pltpu.sync_copy(dego_sh.at[slab], deg_hbm.at[cid, 0, slab])
    pltpu.sync_copy(degi_sh.at[slab], deg_hbm.at[cid, 1, slab])

  return k(ei, zeros1)


def _sc_aggregate(ei, h_scaled, zeros2, n_pad, nchunk, d):
  slab_n = n_pad // NS

  @functools.partial(
      pl.kernel,
      out_type=jax.ShapeDtypeStruct((NC, n_pad, d), jnp.float32),
      mesh=_mesh(),
      scratch_types=[
          pltpu.VMEM((2, nchunk, CHUNK), jnp.int32),
          pltpu.VMEM((CHUNK, d), jnp.float32),
          pltpu.VMEM_SHARED((n_pad, d), jnp.float32),
          pltpu.SemaphoreType.DMA,
      ],
  )
  def k(ei_hbm, h_hbm, z_hbm, agg_hbm, idx_v, rows_v, agg_sh, sem):
    cid = lax.axis_index("c")
    sid = lax.axis_index("s")
    w = cid * NS + sid
    pltpu.sync_copy(ei_hbm.at[0, w], idx_v.at[0])
    pltpu.sync_copy(ei_hbm.at[1, w], idx_v.at[1])
    slab = pl.ds(sid * slab_n, slab_n)
    pltpu.sync_copy(z_hbm.at[slab], agg_sh.at[slab])
    plsc.subcore_barrier()

    def body(j, carry):
      pltpu.async_copy(h_hbm.at[idx_v.at[0, j]], rows_v, sem).wait()
      pltpu.sync_copy(rows_v, agg_sh.at[idx_v.at[1, j]], add=True)
      return carry

    lax.fori_loop(0, nchunk, body, 0)
    plsc.subcore_barrier()
    pltpu.sync_copy(agg_sh.at[slab], agg_hbm.at[cid, slab])

  return k(ei, h_scaled, zeros2)


def _tc_matmul_scale(x_pad, w, degp, n_pad, d):
  grid = n_pad // BLK

  def body(x_ref, w_ref, deg_ref, o_ref):
    deg = deg_ref[0, 0] + deg_ref[1, 0]
    norm = lax.rsqrt(jnp.maximum(deg, 1.0))
    h = jnp.dot(x_ref[...], w_ref[...], preferred_element_type=jnp.float32)
    o_ref[...] = h * norm

  return pl.pallas_call(
      body,
      grid=(grid,),
      in_specs=[
          pl.BlockSpec((BLK, d), lambda i: (i, 0)),
          pl.BlockSpec((d, d), lambda i: (0, 0)),
          pl.BlockSpec((NC, 2, BLK, 1), lambda i: (0, 0, i, 0)),
      ],
      out_specs=pl.BlockSpec((BLK, d), lambda i: (i, 0)),
      out_shape=jax.ShapeDtypeStruct((n_pad, d), jnp.float32),
  )(x_pad, w, degp)


def _tc_finalize(aggp, degp, b2, n_pad, d):
  grid = n_pad // BLK

  def body(agg_ref, deg_ref, b_ref, act_ref, clone_ref):
    agg = agg_ref[0] + agg_ref[1]
    deg = deg_ref[0, 1] + deg_ref[1, 1]
    norm = lax.rsqrt(jnp.maximum(deg, 1.0))
    out = agg * norm + b_ref[...]
    act = jnp.maximum(out, 0.0)
    act_ref[...] = act
    clone_ref[...] = jnp.where(act >= 0.5, 1.0, 0.0).astype(jnp.float32)

  return pl.pallas_call(
      body,
      grid=(grid,),
      in_specs=[
          pl.BlockSpec((NC, BLK, d), lambda i: (0, i, 0)),
          pl.BlockSpec((NC, 2, BLK, 1), lambda i: (0, 0, i, 0)),
          pl.BlockSpec((1, d), lambda i: (0, 0)),
      ],
      out_specs=[
          pl.BlockSpec((BLK, d), lambda i: (i, 0)),
          pl.BlockSpec((BLK, d), lambda i: (i, 0)),
      ],
      out_shape=[
          jax.ShapeDtypeStruct((n_pad, d), jnp.float32),
          jax.ShapeDtypeStruct((n_pad, d), jnp.float32),
      ],
  )(aggp, degp, b2)


def kernel(in_feat, edge_index, W, b):
  n, d = in_feat.shape
  e = edge_index.shape[1]
  ept = e // NW
  assert e % NW == 0 and ept % CHUNK == 0
  nchunk = ept // CHUNK
  n_pad = ((n + BLK - 1) // BLK) * BLK

  ei = edge_index.astype(jnp.int32).reshape(2, NW, nchunk, CHUNK)
  zeros1 = jnp.zeros((n_pad,), jnp.float32)
  zeros2 = jnp.zeros((n_pad, d), jnp.float32)
  x_pad = jnp.pad(in_feat, ((0, n_pad - n), (0, 0)))

  degp = _sc_degrees(ei, zeros1, n_pad, nchunk)
  degp4 = degp.reshape(NC, 2, n_pad, 1)
  h_scaled = _tc_matmul_scale(x_pad, W, degp4, n_pad, d)
  aggp = _sc_aggregate(ei, h_scaled, zeros2, n_pad, nchunk, d)
  h_act, h_clone = _tc_finalize(aggp, degp4, b.reshape(1, d), n_pad, d)
  return (h_act[:n], h_clone[:n])


# speedup vs baseline: 15.4461x; 15.4461x over previous
import functools

import jax
import jax.numpy as jnp
from jax import lax
from jax.experimental import pallas as pl
from jax.experimental.pallas import tpu as pltpu
from jax.experimental.pallas import tpu_sc as plsc

NC = 2
NS = 16
NW = NC * NS
CHUNK = 80
BLK = 2048

_mesh = functools.partial(
    plsc.VectorSubcoreMesh, core_axis_name="c", subcore_axis_name="s",
    num_cores=NC, num_subcores=NS)


def _sc_degrees(ei, zeros1, n_pad, nchunk):
  slab_n = n_pad // NS

  @functools.partial(
      pl.kernel,
      out_type=jax.ShapeDtypeStruct((NC, 2, n_pad), jnp.float32),
      mesh=_mesh(),
      scratch_types=[
          pltpu.VMEM((2, nchunk, CHUNK), jnp.int32),
          pltpu.VMEM((CHUNK,), jnp.float32),
          pltpu.VMEM_SHARED((n_pad,), jnp.float32),
          pltpu.VMEM_SHARED((n_pad,), jnp.float32),
      ],
  )
  def k(ei_hbm, z_hbm, deg_hbm, idx_v, ones_v, dego_sh, degi_sh):
    cid = lax.axis_index("c")
    sid = lax.axis_index("s")
    w = cid * NS + sid
    pltpu.sync_copy(ei_hbm.at[0, w], idx_v.at[0])
    pltpu.sync_copy(ei_hbm.at[1, w], idx_v.at[1])
    for t in range(CHUNK // 16):
      ones_v[pl.ds(t * 16, 16)] = jnp.ones((16,), jnp.float32)
    slab = pl.ds(sid * slab_n, slab_n)
    pltpu.sync_copy(z_hbm.at[slab], dego_sh.at[slab])
    pltpu.sync_copy(z_hbm.at[slab], degi_sh.at[slab])
    plsc.subcore_barrier()

    def body(j, carry):
      pltpu.sync_copy(ones_v, dego_sh.at[idx_v.at[0, j]], add=True)
      pltpu.sync_copy(ones_v, degi_sh.at[idx_v.at[1, j]], add=True)
      return carry

    lax.fori_loop(0, nchunk, body, 0)
    plsc.subcore_barrier()
    pltpu.sync_copy(dego_sh.at[slab], deg_hbm.at[cid, 0, slab])
    pltpu.sync_copy(degi_sh.at[slab], deg_hbm.at[cid, 1, slab])

  return k(ei, zeros1)


def _sc_aggregate(ei, h_scaled, zeros2, n_pad, nchunk, d):
  slab_n = n_pad // NS

  @functools.partial(
      pl.kernel,
      out_type=jax.ShapeDtypeStruct((NC, n_pad, d), jnp.float32),
      mesh=_mesh(),
      scratch_types=[
          pltpu.VMEM((2, nchunk, CHUNK), jnp.int32),
          pltpu.VMEM((CHUNK, d), jnp.float32),
          pltpu.VMEM_SHARED((n_pad, d), jnp.float32),
          pltpu.SemaphoreType.DMA,
      ],
  )
  def k(ei_hbm, h_hbm, z_hbm, agg_hbm, idx_v, rows_v, agg_sh, sem):
    cid = lax.axis_index("c")
    sid = lax.axis_index("s")
    w = cid * NS + sid
    pltpu.sync_copy(ei_hbm.at[0, w], idx_v.at[0])
    pltpu.sync_copy(ei_hbm.at[1, w], idx_v.at[1])
    slab = pl.ds(sid * slab_n, slab_n)
    pltpu.sync_copy(z_hbm.at[slab], agg_sh.at[slab])
    plsc.subcore_barrier()

    def body(j, carry):
      pltpu.async_copy(h_hbm.at[idx_v.at[0, j]], rows_v, sem).wait()
      pltpu.sync_copy(rows_v, agg_sh.at[idx_v.at[1, j]], add=True)
      return carry

    lax.fori_loop(0, nchunk, body, 0)
    plsc.subcore_barrier()
    pltpu.sync_copy(agg_sh.at[slab], agg_hbm.at[cid, slab])

  return k(ei, h_scaled, zeros2)


def _tc_matmul_scale(x_pad, w, degp, n_pad, d):
  grid = n_pad // BLK

  def body(x_ref, w_ref, deg_ref, o_ref):
    deg = deg_ref[0, 0] + deg_ref[1, 0]
    norm = lax.rsqrt(jnp.maximum(deg, 1.0))
    h = jnp.dot(x_ref[...], w_ref[...], preferred_element_type=jnp.float32)
    o_ref[...] = h * norm

  return pl.pallas_call(
      body,
      grid=(grid,),
      in_specs=[
          pl.BlockSpec((BLK, d), lambda i: (i, 0)),
          pl.BlockSpec((d, d), lambda i: (0, 0)),
          pl.BlockSpec((NC, 2, BLK, 1), lambda i: (0, 0, i, 0)),
      ],
      out_specs=pl.BlockSpec((BLK, d), lambda i: (i, 0)),
      out_shape=jax.ShapeDtypeStruct((n_pad, d), jnp.float32),
  )(x_pad, w, degp)


def _tc_finalize(aggp, degp, b2, n_pad, d):
  grid = n_pad // BLK

  def body(agg_ref, deg_ref, b_ref, act_ref, clone_ref):
    agg = agg_ref[0] + agg_ref[1]
    deg = deg_ref[0, 1] + deg_ref[1, 1]
    norm = lax.rsqrt(jnp.maximum(deg, 1.0))
    out = agg * norm + b_ref[...]
    act = jnp.maximum(out, 0.0)
    act_ref[...] = act
    clone_ref[...] = jnp.where(act >= 0.5, 1.0, 0.0).astype(jnp.float32)

  return pl.pallas_call(
      body,
      grid=(grid,),
      in_specs=[
          pl.BlockSpec((NC, BLK, d), lambda i: (0, i, 0)),
          pl.BlockSpec((NC, 2, BLK, 1), lambda i: (0, 0, i, 0)),
          pl.BlockSpec((1, d), lambda i: (0, 0)),
      ],
      out_specs=[
          pl.BlockSpec((BLK, d), lambda i: (i, 0)),
          pl.BlockSpec((BLK, d), lambda i: (i, 0)),
      ],
      out_shape=[
          jax.ShapeDtypeStruct((n_pad, d), jnp.float32),
          jax.ShapeDtypeStruct((n_pad, d), jnp.float32),
      ],
  )(aggp, degp, b2)


def kernel(in_feat, edge_index, W, b):
  n, d = in_feat.shape
  e = edge_index.shape[1]
  ept = e // NW
  assert e % NW == 0 and ept % CHUNK == 0
  nchunk = ept // CHUNK
  n_pad = ((n + BLK - 1) // BLK) * BLK

  ei = edge_index.astype(jnp.int32).reshape(2, NW, nchunk, CHUNK)
  zeros1 = jnp.zeros((n_pad,), jnp.float32)
  zeros2 = jnp.zeros((n_pad, d), jnp.float32)
  x_pad = jnp.pad(in_feat, ((0, n_pad - n), (0, 0)))

  degp = _sc_degrees(ei, zeros1, n_pad, nchunk)
  degp4 = degp.reshape(NC, 2, n_pad, 1)
  h_scaled = _tc_matmul_scale(x_pad, W, degp4, n_pad, d)
  aggp = _sc_aggregate(ei, h_scaled, zeros2, n_pad, nchunk, d)
  h_act, h_clone = _tc_finalize(aggp, degp4, b.reshape(1, d), n_pad, d)
  return (h_act[:n], h_clone[:n])
